# baseline (speedup 1.0000x reference)
import functools

import jax
import jax.numpy as jnp
from jax import lax
from jax.experimental import pallas as pl
from jax.experimental.pallas import tpu as pltpu
from jax.experimental.pallas import tpu_sc as plsc

_NUM_WORKERS = 32
_LANES = 16
_NBUF = 4
_W = 80


def kernel(x, token_emb, pos_emb):
    B, S = x.shape
    V, D = token_emb.shape
    ROWS = B * S
    NBLK = ROWS // (_NUM_WORKERS * _W)
    PROWS = S + _W - (200 % _W or _W)

    x2 = x.reshape(ROWS // _W, _W).astype(jnp.int32)

    mesh = plsc.VectorSubcoreMesh(core_axis_name="c", subcore_axis_name="s")

    @functools.partial(
        pl.kernel,
        mesh=mesh,
        out_type=jax.ShapeDtypeStruct((ROWS, D), jnp.float32),
        scratch_types=[
            pltpu.VMEM((NBLK, _W), jnp.int32),
            pltpu.VMEM((PROWS, D), jnp.float32),
        ] + [pltpu.VMEM((_W, D), jnp.float32) for _ in range(_NBUF)]
          + [pltpu.SemaphoreType.DMA for _ in range(2 * _NBUF)],
    )
    def run(tok_hbm, idx_hbm, pos_hbm, out_hbm, idx_v, pos_v, *rest):
        bufs = rest[:_NBUF]
        gsems = rest[_NBUF:2 * _NBUF]
        wsems = rest[2 * _NBUF:]
        wid = lax.axis_index("s") * 2 + lax.axis_index("c")
        blk_base = wid * NBLK
        pltpu.sync_copy(idx_hbm.at[pl.ds(blk_base, NBLK)], idx_v)
        pltpu.sync_copy(pos_hbm, pos_v.at[pl.ds(0, S)])
        pltpu.sync_copy(pos_hbm.at[pl.ds(0, PROWS - S)],
                        pos_v.at[pl.ds(S, PROWS - S)])

        def issue_gather(blk, j):
            pltpu.async_copy(tok_hbm.at[idx_v.at[blk]], bufs[j], gsems[j])

        def wait_gather(blk, j):
            pltpu.make_async_copy(tok_hbm.at[idx_v.at[blk]], bufs[j],
                                  gsems[j]).wait()

        def issue_writeback(blk, j):
            pltpu.async_copy(bufs[j],
                             out_hbm.at[pl.ds((blk_base + blk) * _W, _W)],
                             wsems[j])

        def wait_writeback(j):
            pltpu.make_async_copy(bufs[j], out_hbm.at[pl.ds(0, _W)],
                                  wsems[j]).wait()

        def add_pos(blk, j):
            buf = bufs[j]
            phase = lax.rem(blk * _W, S)

            @pl.loop(0, _W)
            def _(r):
                pr = phase + r
                for c in range(D // _LANES):
                    sl = pl.ds(c * _LANES, _LANES)
                    plsc.addupdate(buf.at[r, sl], pos_v[pr, sl])

        for j in range(_NBUF - 1):
            issue_gather(j, j)

        @pl.loop(0, NBLK // _NBUF)
        def _(t):
            for b in range(_NBUF):
                blk = _NBUF * t + b
                jg = (b + _NBUF - 1) % _NBUF
                blk_g = blk + _NBUF - 1

                @pl.when(blk_g < NBLK)
                def _():
                    @pl.when(blk_g >= _NBUF)
                    def _():
                        wait_writeback(jg)
                    issue_gather(blk_g, jg)

                wait_gather(blk, b)
                add_pos(blk, b)
                issue_writeback(blk, b)

        for j in range(_NBUF):
            wait_writeback(j)

    out = run(token_emb, x2, pos_emb)
    return out.reshape(B, S, D)


# speedup vs baseline: 2.6022x; 2.6022x over previous
import functools

import jax
import jax.numpy as jnp
from jax import lax
from jax.experimental import pallas as pl
from jax.experimental.pallas import tpu as pltpu
from jax.experimental.pallas import tpu_sc as plsc

_NUM_WORKERS = 32
_LANES = 16


def kernel(x, token_emb, pos_emb):
    B, S = x.shape
    V, D = token_emb.shape
    HALF = S // 2
    SEQ_PER_W = B // _NUM_WORKERS

    x2 = x.reshape(B * 2, HALF).astype(jnp.int32)

    mesh = plsc.VectorSubcoreMesh(core_axis_name="c", subcore_axis_name="s")

    @functools.partial(
        pl.kernel,
        mesh=mesh,
        out_type=jax.ShapeDtypeStruct((B * S, D), jnp.float32),
        scratch_types=[
            pltpu.VMEM((2 * SEQ_PER_W, HALF), jnp.int32),
            pltpu.VMEM((S, D), jnp.float32),
            pltpu.VMEM((S, D), jnp.float32),
            pltpu.VMEM((S, D), jnp.float32),
            pltpu.SemaphoreType.DMA,
            pltpu.SemaphoreType.DMA,
            pltpu.SemaphoreType.DMA,
            pltpu.SemaphoreType.DMA,
            pltpu.SemaphoreType.DMA,
            pltpu.SemaphoreType.DMA,
        ],
    )
    def run(tok_hbm, idx_hbm, pos_hbm, out_hbm, idx_v, pos_v,
            buf0, buf1, g00, g01, g10, g11, wsem0, wsem1):
        bufs = (buf0, buf1)
        gsems = ((g00, g01), (g10, g11))
        wsems = (wsem0, wsem1)
        wid = lax.axis_index("s") * 2 + lax.axis_index("c")
        seq_base = wid * SEQ_PER_W
        pltpu.sync_copy(idx_hbm.at[pl.ds(seq_base * 2, 2 * SEQ_PER_W)], idx_v)
        pltpu.sync_copy(pos_hbm, pos_v)

        def half_copy(blk, b, h):
            return pltpu.make_async_copy(
                tok_hbm.at[idx_v.at[2 * blk + h]],
                bufs[b].at[pl.ds(h * HALF, HALF)], gsems[b][h])

        def issue_gather(blk, b):
            half_copy(blk, b, 0).start()
            half_copy(blk, b, 1).start()

        def issue_writeback(blk, b):
            pltpu.async_copy(bufs[b], out_hbm.at[pl.ds((seq_base + blk) * S, S)],
                             wsems[b])

        def wait_writeback(b):
            pltpu.make_async_copy(bufs[b], out_hbm.at[pl.ds(0, S)],
                                  wsems[b]).wait()

        def add_pos_half(b, h):
            buf = bufs[b]

            @pl.loop(h * HALF, (h + 1) * HALF)
            def _(r):
                for c in range(D // _LANES):
                    sl = pl.ds(c * _LANES, _LANES)
                    plsc.addupdate(buf.at[r, sl], pos_v[r, sl])

        def process(blk, b):
            half_copy(blk, b, 0).wait()
            add_pos_half(b, 0)
            half_copy(blk, b, 1).wait()
            add_pos_half(b, 1)
            issue_writeback(blk, b)

        issue_gather(0, 0)

        @pl.loop(0, SEQ_PER_W // 2)
        def _(t):
            for b in range(2):
                blk = 2 * t + b
                if b == 0:
                    @pl.when(t > 0)
                    def _():
                        wait_writeback(1)
                        issue_gather(blk + 1, 1)

                    @pl.when(t == 0)
                    def _():
                        issue_gather(blk + 1, 1)
                else:
                    @pl.when(blk + 1 < SEQ_PER_W)
                    def _():
                        wait_writeback(0)
                        issue_gather(blk + 1, 0)
                process(blk, b)

        wait_writeback(0)
        wait_writeback(1)

    out = run(token_emb, x2, pos_emb)
    return out.reshape(B, S, D)


# speedup vs baseline: 2.6041x; 1.0007x over previous
import functools

import jax
import jax.numpy as jnp
from jax import lax
from jax.experimental import pallas as pl
from jax.experimental.pallas import tpu as pltpu
from jax.experimental.pallas import tpu_sc as plsc

_NUM_WORKERS = 32
_LANES = 16
_NBUF = 3
_IDXSTAGE = 136


def kernel(x, token_emb, pos_emb):
    B, S = x.shape
    V, D = token_emb.shape
    HALF = S // 2
    NBLK = B // _NUM_WORKERS
    SPLIT = _IDXSTAGE // 2

    x2 = x.reshape(B * 2, HALF).astype(jnp.int32)

    mesh = plsc.VectorSubcoreMesh(core_axis_name="c", subcore_axis_name="s")

    @functools.partial(
        pl.kernel,
        mesh=mesh,
        out_type=jax.ShapeDtypeStruct((B * S, D), jnp.float32),
        scratch_types=[
            pltpu.VMEM((_IDXSTAGE, HALF), jnp.int32),
            pltpu.VMEM((S, D), jnp.float32),
        ] + [pltpu.VMEM((S, D), jnp.float32) for _ in range(_NBUF)]
          + [pltpu.SemaphoreType.DMA for _ in range(2 * _NBUF + 1)],
    )
    def run(tok_hbm, idx_hbm, pos_hbm, out_hbm, idx_v, pos_v, *rest):
        bufs = rest[:_NBUF]
        gsems = rest[_NBUF:2 * _NBUF]
        wsems = rest[2 * _NBUF:3 * _NBUF]
        rsem = rest[3 * _NBUF]
        wid = lax.axis_index("s") * 2 + lax.axis_index("c")
        seq_base = wid * NBLK
        pltpu.sync_copy(idx_hbm.at[pl.ds(seq_base * 2, _IDXSTAGE)], idx_v)
        pltpu.sync_copy(pos_hbm, pos_v)

        RELOAD_ROWS = 2 * NBLK - _IDXSTAGE

        def reload_copy():
            return pltpu.make_async_copy(
                idx_hbm.at[pl.ds(seq_base * 2 + _IDXSTAGE, RELOAD_ROWS)],
                idx_v.at[pl.ds(0, RELOAD_ROWS)], rsem)

        def idx_row(blk):
            return 2 * blk - jnp.where(blk >= SPLIT, _IDXSTAGE, 0)

        def issue_gather(blk, j):
            off = idx_row(blk)
            pltpu.async_copy(tok_hbm.at[idx_v.at[off]],
                             bufs[j].at[pl.ds(0, HALF)], gsems[j])
            pltpu.async_copy(tok_hbm.at[idx_v.at[off + 1]],
                             bufs[j].at[pl.ds(HALF, HALF)], gsems[j])

        def wait_gather(blk, j):
            off = idx_row(blk)
            pltpu.make_async_copy(tok_hbm.at[idx_v.at[off]],
                                  bufs[j].at[pl.ds(0, HALF)], gsems[j]).wait()
            pltpu.make_async_copy(tok_hbm.at[idx_v.at[off + 1]],
                                  bufs[j].at[pl.ds(HALF, HALF)],
                                  gsems[j]).wait()

        def issue_writeback(blk, j):
            pltpu.async_copy(bufs[j],
                             out_hbm.at[pl.ds((seq_base + blk) * S, S)],
                             wsems[j])

        def wait_writeback(j):
            pltpu.make_async_copy(bufs[j], out_hbm.at[pl.ds(0, S)],
                                  wsems[j]).wait()

        def add_pos(j):
            buf = bufs[j]

            @pl.loop(0, S)
            def _(r):
                for c in range(D // _LANES):
                    sl = pl.ds(c * _LANES, _LANES)
                    plsc.addupdate(buf.at[r, sl], pos_v[r, sl])

        for j in range(_NBUF - 1):
            issue_gather(j, j)

        NT = (NBLK + _NBUF - 1) // _NBUF

        @pl.loop(0, NT)
        def _(t):
            for b in range(_NBUF):
                blk = _NBUF * t + b
                jg = (b + _NBUF - 1) % _NBUF
                blk_g = blk + _NBUF - 1

                @pl.when(blk == SPLIT - 8)
                def _():
                    reload_copy().start()

                @pl.when(blk_g < NBLK)
                def _():
                    @pl.when(blk >= 1)
                    def _():
                        wait_writeback(jg)

                    @pl.when(blk_g == SPLIT)
                    def _():
                        reload_copy().wait()
                    issue_gather(blk_g, jg)

                @pl.when(blk < NBLK)
                def _():
                    wait_gather(blk, b)
                    add_pos(b)
                    issue_writeback(blk, b)

        for j in range(_NBUF):
            wait_writeback(j)

    out = run(token_emb, x2, pos_emb)
    return out.reshape(B, S, D)
